# Initial kernel scaffold; baseline (speedup 1.0000x reference)
import jax
import jax.numpy as jnp
from jax.experimental import pallas as pl


def kernel(inputs, targets, alpha):
    raise NotImplementedError("write your pallas kernel here")


# speedup vs baseline: 1.9637x; 1.9637x over previous
import functools

import jax
import jax.numpy as jnp
from jax import lax
from jax.experimental import pallas as pl
from jax.experimental.pallas import tpu as pltpu
from jax.experimental.pallas import tpu_sc as plsc

N = 16384
C = 1000
L = 16
NC = 2
NS = 16
NW = NC * NS
ROWS_PER = N // NW
DMA_CH = 4
CH_W = ROWS_PER // DMA_CH


def _sc_partials(inputs_flat, targets_2d, alpha_flat):
    mesh = plsc.VectorSubcoreMesh(core_axis_name="c", subcore_axis_name="s")

    @functools.partial(
        pl.kernel,
        mesh=mesh,
        out_type=jax.ShapeDtypeStruct((NW, L), jnp.float32),
        scratch_types=[
            pltpu.VMEM((DMA_CH, CH_W), jnp.int32),
            pltpu.VMEM((DMA_CH, CH_W), jnp.int32),
            pltpu.VMEM((DMA_CH, CH_W), jnp.float32),
            pltpu.VMEM((DMA_CH, CH_W), jnp.float32),
            pltpu.VMEM((L,), jnp.float32),
            pltpu.SemaphoreType.DMA,
        ],
    )
    def sc_kernel(inp_hbm, tgt_hbm, alpha_hbm, out_hbm,
                  t_v, idx_v, x_v, a_v, acc_v, sem):
        cid = lax.axis_index("c")
        sid = lax.axis_index("s")
        wid = sid * NC + cid
        base = wid * ROWS_PER

        pltpu.sync_copy(tgt_hbm.at[pl.ds(wid * DMA_CH, DMA_CH)], t_v)

        alpha_cp = [
            pltpu.async_copy(alpha_hbm.at[t_v.at[j]], a_v.at[j], sem)
            for j in range(DMA_CH)
        ]

        for j in range(DMA_CH):
            for i in range(CH_W // L):
                k = j * (CH_W // L) + i
                t = t_v[j, pl.ds(i * L, L)]
                row = lax.iota(jnp.int32, L) + (base + k * L)
                idx_v[j, pl.ds(i * L, L)] = row * C + t

        logit_cp = [
            pltpu.async_copy(inp_hbm.at[idx_v.at[j]], x_v.at[j], sem)
            for j in range(DMA_CH)
        ]
        for cp in alpha_cp + logit_cp:
            cp.wait()

        acc = jnp.zeros((L,), jnp.float32)
        for j in range(DMA_CH):
            for i in range(CH_W // L):
                x = x_v[j, pl.ds(i * L, L)]
                a = a_v[j, pl.ds(i * L, L)]
                om = 1.0 - jnp.exp(x)
                acc = acc - a * om * om * x
        acc_v[...] = acc * (1.0 / N)
        pltpu.sync_copy(acc_v, out_hbm.at[wid])

    return sc_kernel(inputs_flat, targets_2d, alpha_flat)


def _tc_sum(partials):
    def body(x_ref, o_ref):
        o_ref[0, 0] = jnp.sum(x_ref[...])

    return pl.pallas_call(
        body,
        out_shape=jax.ShapeDtypeStruct((1, 1), jnp.float32),
        out_specs=pl.BlockSpec(memory_space=pltpu.SMEM),
    )(partials)


def kernel(inputs, targets, alpha):
    inputs_flat = inputs.reshape(-1)
    targets_2d = targets.astype(jnp.int32).reshape(NW * DMA_CH, CH_W)
    alpha_flat = alpha.reshape(-1).astype(jnp.float32)
    partials = _sc_partials(inputs_flat, targets_2d, alpha_flat)
    return _tc_sum(partials)[0, 0]


# speedup vs baseline: 2.5644x; 1.3059x over previous
import functools

import jax
import jax.numpy as jnp
from jax import lax
from jax.experimental import pallas as pl
from jax.experimental.pallas import tpu as pltpu
from jax.experimental.pallas import tpu_sc as plsc

N = 16384
C = 1000
NC = 2
NS = 16
NW = NC * NS
ROWS_PER = N // NW
DMA_CH = 4
CH_W = ROWS_PER // DMA_CH

BR = 1024
G = N // BR


def _sc_alpha_gather(targets_2d, alpha_flat):
    mesh = plsc.VectorSubcoreMesh(core_axis_name="c", subcore_axis_name="s")

    @functools.partial(
        pl.kernel,
        mesh=mesh,
        out_type=jax.ShapeDtypeStruct((N,), jnp.float32),
        scratch_types=[
            pltpu.VMEM((DMA_CH, CH_W), jnp.int32),
            pltpu.VMEM((DMA_CH, CH_W), jnp.float32),
            pltpu.SemaphoreType.DMA,
        ],
    )
    def sc_kernel(tgt_hbm, alpha_hbm, out_hbm, t_v, a_v, sem):
        cid = lax.axis_index("c")
        sid = lax.axis_index("s")
        wid = sid * NC + cid
        pltpu.sync_copy(tgt_hbm.at[pl.ds(wid * DMA_CH, DMA_CH)], t_v)
        copies = [
            pltpu.async_copy(alpha_hbm.at[t_v.at[j]], a_v.at[j], sem)
            for j in range(DMA_CH)
        ]
        for cp in copies:
            cp.wait()
        base = wid * ROWS_PER
        for j in range(DMA_CH):
            pltpu.sync_copy(a_v.at[j], out_hbm.at[pl.ds(base + j * CH_W, CH_W)])

    return sc_kernel(targets_2d, alpha_flat)


def _tc_focal(inputs, targets_col, alpha_col):
    def body(x_ref, t_ref, a_ref, o_ref, acc_ref):
        j = pl.program_id(0)

        @pl.when(j == 0)
        def _init():
            acc_ref[0] = 0.0

        x = x_ref[...]
        t = t_ref[...]
        cls = lax.broadcasted_iota(jnp.int32, (BR, C), 1)
        xg = jnp.sum(jnp.where(cls == t, x, 0.0), axis=1, keepdims=True)
        om = 1.0 - jnp.exp(xg)
        term = a_ref[...] * om * om * xg
        acc_ref[0] += jnp.sum(term)

        @pl.when(j == G - 1)
        def _fin():
            o_ref[0, 0] = -acc_ref[0] * (1.0 / N)

    return pl.pallas_call(
        body,
        grid=(G,),
        in_specs=[
            pl.BlockSpec((BR, C), lambda j: (j, 0)),
            pl.BlockSpec((BR, 1), lambda j: (j, 0)),
            pl.BlockSpec((BR, 1), lambda j: (j, 0)),
        ],
        out_specs=pl.BlockSpec((1, 1), lambda j: (0, 0), memory_space=pltpu.SMEM),
        out_shape=jax.ShapeDtypeStruct((1, 1), jnp.float32),
        scratch_shapes=[pltpu.SMEM((1,), jnp.float32)],
    )(inputs, targets_col, alpha_col)


def kernel(inputs, targets, alpha):
    tgt = targets.astype(jnp.int32)
    alpha_flat = alpha.reshape(-1).astype(jnp.float32)
    a_sel = _sc_alpha_gather(tgt.reshape(NW * DMA_CH, CH_W), alpha_flat)
    out = _tc_focal(inputs, tgt.reshape(N, 1), a_sel.reshape(N, 1))
    return out[0, 0]


# speedup vs baseline: 2.7131x; 1.0580x over previous
import functools

import jax
import jax.numpy as jnp
from jax import lax
from jax.experimental import pallas as pl
from jax.experimental.pallas import tpu as pltpu
from jax.experimental.pallas import tpu_sc as plsc

N = 16384
C = 1000
NC = 2
NS = 16
NW = NC * NS
ROWS_PER = N // NW
DMA_CH = 4
CH_W = ROWS_PER // DMA_CH

BR = 1024
G = N // BR
SUB = BR // 128


def _sc_alpha_gather(targets_2d, alpha_flat):
    mesh = plsc.VectorSubcoreMesh(core_axis_name="c", subcore_axis_name="s")

    @functools.partial(
        pl.kernel,
        mesh=mesh,
        out_type=jax.ShapeDtypeStruct((N // CH_W, CH_W), jnp.float32),
        scratch_types=[
            pltpu.VMEM((DMA_CH, CH_W), jnp.int32),
            pltpu.VMEM((DMA_CH, CH_W), jnp.float32),
            pltpu.SemaphoreType.DMA,
        ],
    )
    def sc_kernel(tgt_hbm, alpha_hbm, out_hbm, t_v, a_v, sem):
        cid = lax.axis_index("c")
        sid = lax.axis_index("s")
        wid = sid * NC + cid
        pltpu.sync_copy(tgt_hbm.at[pl.ds(wid * DMA_CH, DMA_CH)], t_v)
        copies = [
            pltpu.async_copy(alpha_hbm.at[t_v.at[j]], a_v.at[j], sem)
            for j in range(DMA_CH)
        ]
        for cp in copies:
            cp.wait()
        pltpu.sync_copy(a_v, out_hbm.at[pl.ds(wid * DMA_CH, DMA_CH)])

    return sc_kernel(targets_2d, alpha_flat)


def _tc_focal(inputs, targets_2d, alpha_2d):
    def body(x_ref, t_ref, a_ref, o_ref, acc_ref):
        j = pl.program_id(0)

        @pl.when(j == 0)
        def _init():
            acc_ref[0] = 0.0

        t_cols = t_ref[...].T
        a_cols = a_ref[...].T
        cls = lax.broadcasted_iota(jnp.int32, (128, C), 1)
        part = jnp.float32(0.0)
        for r in range(SUB):
            xs = x_ref[r * 128:(r + 1) * 128, :]
            t = t_cols[:, r:r + 1]
            a = a_cols[:, r:r + 1]
            xg = jnp.sum(jnp.where(cls == t, xs, 0.0), axis=1, keepdims=True)
            om = 1.0 - jnp.exp(xg)
            part += jnp.sum(a * om * om * xg)
        acc_ref[0] += part

        @pl.when(j == G - 1)
        def _fin():
            o_ref[0, 0] = -acc_ref[0] * (1.0 / N)

    return pl.pallas_call(
        body,
        grid=(G,),
        in_specs=[
            pl.BlockSpec((BR, C), lambda j: (j, 0)),
            pl.BlockSpec((SUB, 128), lambda j: (j, 0)),
            pl.BlockSpec((SUB, 128), lambda j: (j, 0)),
        ],
        out_specs=pl.BlockSpec((1, 1), lambda j: (0, 0), memory_space=pltpu.SMEM),
        out_shape=jax.ShapeDtypeStruct((1, 1), jnp.float32),
        scratch_shapes=[pltpu.SMEM((1,), jnp.float32)],
    )(inputs, targets_2d, alpha_2d)


def kernel(inputs, targets, alpha):
    tgt2d = targets.astype(jnp.int32).reshape(N // CH_W, CH_W)
    alpha_flat = alpha.reshape(-1).astype(jnp.float32)
    a_sel = _sc_alpha_gather(tgt2d, alpha_flat)
    out = _tc_focal(inputs, tgt2d, a_sel)
    return out[0, 0]


# speedup vs baseline: 5.6743x; 2.0914x over previous
import functools

import jax
import jax.numpy as jnp
from jax import lax
from jax.experimental import pallas as pl
from jax.experimental.pallas import tpu as pltpu
from jax.experimental.pallas import tpu_sc as plsc

N = 16384
C = 1000
NC = 2
NS = 16
NW = NC * NS
ROWS_PER = N // NW
DMA_CH = 4
CH_W = ROWS_PER // DMA_CH

BR = 1024
G = N // BR


def _sc_alpha_gather(targets_2d, alpha_flat):
    mesh = plsc.VectorSubcoreMesh(core_axis_name="c", subcore_axis_name="s")

    @functools.partial(
        pl.kernel,
        mesh=mesh,
        out_type=jax.ShapeDtypeStruct((N // CH_W, CH_W), jnp.float32),
        scratch_types=[
            pltpu.VMEM((DMA_CH, CH_W), jnp.int32),
            pltpu.VMEM((DMA_CH, CH_W), jnp.float32),
            pltpu.SemaphoreType.DMA,
        ],
    )
    def sc_kernel(tgt_hbm, alpha_hbm, out_hbm, t_v, a_v, sem):
        cid = lax.axis_index("c")
        sid = lax.axis_index("s")
        wid = sid * NC + cid
        pltpu.sync_copy(tgt_hbm.at[pl.ds(wid * DMA_CH, DMA_CH)], t_v)
        copies = [
            pltpu.async_copy(alpha_hbm.at[t_v.at[j]], a_v.at[j], sem)
            for j in range(DMA_CH)
        ]
        for cp in copies:
            cp.wait()
        pltpu.sync_copy(a_v, out_hbm.at[pl.ds(wid * DMA_CH, DMA_CH)])

    return sc_kernel(targets_2d, alpha_flat)


def _tc_focal(inputs_t, targets_3d, alpha_3d):
    def body(x_ref, t_ref, a_ref, o_ref, acc_ref):
        j = pl.program_id(0)

        @pl.when(j == 0)
        def _init():
            acc_ref[0] = 0.0

        xb = x_ref[...]
        t = t_ref[0]
        a = a_ref[0]
        cls = lax.broadcasted_iota(jnp.int32, (C, BR), 0)
        xg = jnp.sum(jnp.where(cls == t, xb, 0.0), axis=0, keepdims=True)
        om = 1.0 - jnp.exp(xg)
        acc_ref[0] += jnp.sum(a * om * om * xg)

        @pl.when(j == G - 1)
        def _fin():
            o_ref[0, 0] = -acc_ref[0] * (1.0 / N)

    return pl.pallas_call(
        body,
        grid=(G,),
        in_specs=[
            pl.BlockSpec((C, BR), lambda j: (0, j)),
            pl.BlockSpec((1, 1, BR), lambda j: (j, 0, 0)),
            pl.BlockSpec((1, 1, BR), lambda j: (j, 0, 0)),
        ],
        out_specs=pl.BlockSpec((1, 1), lambda j: (0, 0), memory_space=pltpu.SMEM),
        out_shape=jax.ShapeDtypeStruct((1, 1), jnp.float32),
        scratch_shapes=[pltpu.SMEM((1,), jnp.float32)],
    )(inputs_t, targets_3d, alpha_3d)


def kernel(inputs, targets, alpha):
    tgt = targets.astype(jnp.int32)
    alpha_flat = alpha.reshape(-1).astype(jnp.float32)
    a_sel = _sc_alpha_gather(tgt.reshape(N // CH_W, CH_W), alpha_flat)
    out = _tc_focal(
        inputs.T,
        tgt.reshape(G, 1, BR),
        a_sel.reshape(G, 1, BR),
    )
    return out[0, 0]


# speedup vs baseline: 6.1614x; 1.0859x over previous
import functools

import jax
import jax.numpy as jnp
from jax import lax
from jax.experimental import pallas as pl
from jax.experimental.pallas import tpu as pltpu
from jax.experimental.pallas import tpu_sc as plsc

N = 16384
C = 1000
NC = 2
NS = 16
NW = NC * NS
ROWS_PER = N // NW
DMA_CH = 4
CH_W = ROWS_PER // DMA_CH

BR = 2048
G = N // BR


def _sc_alpha_gather(targets_2d, alpha_flat):
    mesh = plsc.VectorSubcoreMesh(core_axis_name="c", subcore_axis_name="s")

    @functools.partial(
        pl.kernel,
        mesh=mesh,
        out_type=jax.ShapeDtypeStruct((N // CH_W, CH_W), jnp.float32),
        scratch_types=[
            pltpu.VMEM((DMA_CH, CH_W), jnp.int32),
            pltpu.VMEM((DMA_CH, CH_W), jnp.float32),
            pltpu.SemaphoreType.DMA,
        ],
    )
    def sc_kernel(tgt_hbm, alpha_hbm, out_hbm, t_v, a_v, sem):
        cid = lax.axis_index("c")
        sid = lax.axis_index("s")
        wid = sid * NC + cid
        pltpu.sync_copy(tgt_hbm.at[pl.ds(wid * DMA_CH, DMA_CH)], t_v)
        copies = [
            pltpu.async_copy(alpha_hbm.at[t_v.at[j]], a_v.at[j], sem)
            for j in range(DMA_CH)
        ]
        for cp in copies:
            cp.wait()
        pltpu.sync_copy(a_v, out_hbm.at[pl.ds(wid * DMA_CH, DMA_CH)])

    return sc_kernel(targets_2d, alpha_flat)


def _tc_g(inputs_t, targets_3d):
    def body(x_ref, t_ref, g_ref):
        xb = x_ref[...]
        t = t_ref[0]
        cls = lax.broadcasted_iota(jnp.int32, (C, BR), 0)
        xg = jnp.sum(jnp.where(cls == t, xb, 0.0), axis=0, keepdims=True)
        om = 1.0 - jnp.exp(xg)
        g_ref[0] = om * om * xg

    return pl.pallas_call(
        body,
        grid=(G,),
        in_specs=[
            pl.BlockSpec((C, BR), lambda j: (0, j)),
            pl.BlockSpec((1, 1, BR), lambda j: (j, 0, 0)),
        ],
        out_specs=pl.BlockSpec((1, 1, BR), lambda j: (j, 0, 0)),
        out_shape=jax.ShapeDtypeStruct((G, 1, BR), jnp.float32),
    )(inputs_t, targets_3d)


def _tc_combine(g3, a3):
    def body(g_ref, a_ref, o_ref):
        o_ref[0, 0] = -jnp.sum(g_ref[...] * a_ref[...]) * (1.0 / N)

    return pl.pallas_call(
        body,
        out_specs=pl.BlockSpec(memory_space=pltpu.SMEM),
        out_shape=jax.ShapeDtypeStruct((1, 1), jnp.float32),
    )(g3, a3)


def kernel(inputs, targets, alpha):
    tgt = targets.astype(jnp.int32)
    alpha_flat = alpha.reshape(-1).astype(jnp.float32)
    a_sel = _sc_alpha_gather(tgt.reshape(N // CH_W, CH_W), alpha_flat)
    g3 = _tc_g(inputs.T, tgt.reshape(G, 1, BR))
    out = _tc_combine(g3, a_sel.reshape(G, 1, BR))
    return out[0, 0]
